# Initial kernel scaffold; baseline (speedup 1.0000x reference)
import jax
import jax.numpy as jnp
from jax.experimental import pallas as pl


def kernel(fixed_table, trainable_table, regular_table, x, vocab_to_custom, vocab_to_regular):
    raise NotImplementedError("write your pallas kernel here")


# speedup vs baseline: 2.2449x; 2.2449x over previous
import functools
import jax
import jax.numpy as jnp
from jax import lax
from jax.experimental import pallas as pl
from jax.experimental.pallas import tpu as pltpu
from jax.experimental.pallas import tpu_sc as plsc

DIM = 64
NUM_CORES = 2
NUM_SUBCORES = 16
NUM_WORKERS = NUM_CORES * NUM_SUBCORES
CHUNK = 128


def _sc_lookup(fixed_table, trainable_table, regular_table, x_flat, v2c, v2r):
    n = x_flat.shape[0]
    per_w = n // NUM_WORKERS
    n_chunks = per_w // CHUNK
    mesh = plsc.VectorSubcoreMesh(core_axis_name="c", subcore_axis_name="s")

    @functools.partial(
        pl.kernel,
        out_type=jax.ShapeDtypeStruct((n, DIM), jnp.float32),
        mesh=mesh,
        compiler_params=pltpu.CompilerParams(use_tc_tiling_on_sc=False),
        scratch_types=[
            pltpu.VMEM((CHUNK,), jnp.int32),
            pltpu.VMEM((CHUNK,), jnp.int32),
            pltpu.VMEM((CHUNK,), jnp.int32),
            pltpu.VMEM((CHUNK, DIM), jnp.float32),
            pltpu.VMEM((CHUNK, DIM), jnp.float32),
            pltpu.VMEM((CHUNK, DIM), jnp.float32),
            pltpu.SemaphoreType.DMA,
            pltpu.SemaphoreType.DMA,
            pltpu.SemaphoreType.DMA,
        ],
    )
    def body(fixed_h, train_h, reg_h, x_h, v2c_h, v2r_h, out_h,
             xv, cv, rv, acc, tmp_b, tmp_c, s0, s1, s2):
        wid = lax.axis_index("s") * NUM_CORES + lax.axis_index("c")
        base_w = wid * per_w

        def chunk_body(g, carry):
            base = base_w + g * CHUNK
            pltpu.sync_copy(x_h.at[pl.ds(base, CHUNK)], xv)
            d0 = pltpu.async_copy(v2c_h.at[xv], cv, s0)
            d1 = pltpu.async_copy(v2r_h.at[xv], rv, s1)
            d0.wait()
            d1.wait()
            g0 = pltpu.async_copy(fixed_h.at[cv], acc, s0)
            g1 = pltpu.async_copy(train_h.at[cv], tmp_b, s1)
            g2 = pltpu.async_copy(reg_h.at[rv], tmp_c, s2)
            g0.wait()
            g1.wait()
            g2.wait()

            def add_row(i, c2):
                for q in range(DIM // 16):
                    sl = pl.ds(q * 16, 16)
                    acc[i, sl] = acc[i, sl] + tmp_b[i, sl] + tmp_c[i, sl]
                return c2

            lax.fori_loop(0, CHUNK, add_row, 0, unroll=2)
            pltpu.sync_copy(acc, out_h.at[pl.ds(base, CHUNK)])
            return carry

        lax.fori_loop(0, n_chunks, chunk_body, 0)

    return body(fixed_table, trainable_table, regular_table, x_flat, v2c, v2r)


def kernel(fixed_table, trainable_table, regular_table, x, vocab_to_custom, vocab_to_regular):
    b, l = x.shape
    x_flat = jnp.reshape(x, (b * l,)).astype(jnp.int32)
    v2c = vocab_to_custom.astype(jnp.int32)
    v2r = vocab_to_regular.astype(jnp.int32)
    out = _sc_lookup(fixed_table, trainable_table, regular_table, x_flat, v2c, v2r)
    return jnp.reshape(out, (b, l, DIM))
